# Initial kernel scaffold; baseline (speedup 1.0000x reference)
import jax
import jax.numpy as jnp
from jax.experimental import pallas as pl


def kernel(z, edge_index, W, b):
    raise NotImplementedError("write your pallas kernel here")


# speedup vs baseline: 21.1268x; 21.1268x over previous
import functools

import jax
import jax.numpy as jnp
from jax import lax
from jax.experimental import pallas as pl
from jax.experimental.pallas import tpu as pltpu
from jax.experimental.pallas import tpu_sc as plsc

NC, NS, L = 2, 16, 16
N_NODES = 10000
D = 128
N_EDGES = 320000
NP = 10240
EP = 327680
ROWS = EP // 128
RPW = ROWS // (NC * NS)
NPT = NP // NS

_sc_mesh = plsc.VectorSubcoreMesh(core_axis_name="c", subcore_axis_name="s")


def _deg_body(dst_hbm, deg_hbm, idx_v, ones_v, buf_v, deg_sh, sem):
    c = lax.axis_index("c")
    s = lax.axis_index("s")
    row0 = (c * NS + s) * RPW
    for i in range(128 // 16):
        ones_v[pl.ds(i * 16, 16)] = jnp.ones((16,), jnp.float32)
    for i in range(NPT // 16):
        buf_v[pl.ds(i * 16, 16)] = jnp.zeros((16,), jnp.float32)
    pltpu.sync_copy(buf_v, deg_sh.at[pl.ds(s * NPT, NPT)])
    pltpu.sync_copy(dst_hbm.at[pl.ds(row0, RPW)], idx_v)
    plsc.subcore_barrier()

    def step(j, carry):
        pltpu.sync_copy(ones_v, deg_sh.at[idx_v.at[j]], add=True)
        return carry

    lax.fori_loop(0, RPW, step, 0)
    plsc.subcore_barrier()
    pltpu.sync_copy(deg_sh.at[pl.ds(s * NPT, NPT)], buf_v)
    pltpu.sync_copy(buf_v, deg_hbm.at[c, pl.ds(s * NPT, NPT)])


_deg_call = pl.kernel(
    _deg_body,
    out_type=jax.ShapeDtypeStruct((NC, NP), jnp.float32),
    mesh=_sc_mesh,
    scratch_types=[
        pltpu.VMEM((RPW, 128), jnp.int32),
        pltpu.VMEM((128,), jnp.float32),
        pltpu.VMEM((NPT,), jnp.float32),
        pltpu.VMEM_SHARED((NP,), jnp.float32),
        pltpu.SemaphoreType.DMA,
    ],
)


def _agg_body(src_hbm, dst_hbm, h2_hbm, acc_hbm, sidx, didx, rows_v, zrow, acc_sh, sem):
    c = lax.axis_index("c")
    s = lax.axis_index("s")
    row0 = (c * NS + s) * RPW
    pltpu.sync_copy(h2_hbm.at[pl.ds(NP - 64, 64)], zrow)
    for t in range(NPT // 64):
        pltpu.sync_copy(zrow, acc_sh.at[pl.ds(s * NPT + t * 64, 64)])
    pltpu.sync_copy(src_hbm.at[pl.ds(row0, RPW)], sidx)
    pltpu.sync_copy(dst_hbm.at[pl.ds(row0, RPW)], didx)
    plsc.subcore_barrier()

    def step(j, carry):
        pltpu.async_copy(h2_hbm.at[sidx.at[j]], rows_v, sem).wait()
        pltpu.sync_copy(rows_v, acc_sh.at[didx.at[j]], add=True)
        return carry

    lax.fori_loop(0, RPW, step, 0)
    plsc.subcore_barrier()
    for t in range(NPT // 128):
        pltpu.sync_copy(acc_sh.at[pl.ds(s * NPT + t * 128, 128)], rows_v)
        pltpu.sync_copy(rows_v, acc_hbm.at[c, pl.ds(s * NPT + t * 128, 128)])


_agg_call = pl.kernel(
    _agg_body,
    out_type=jax.ShapeDtypeStruct((NC, NP, D), jnp.float32),
    mesh=_sc_mesh,
    scratch_types=[
        pltpu.VMEM((RPW, 128), jnp.int32),
        pltpu.VMEM((RPW, 128), jnp.int32),
        pltpu.VMEM((128, D), jnp.float32),
        pltpu.VMEM((64, D), jnp.float32),
        pltpu.VMEM_SHARED((NP, D), jnp.float32),
        pltpu.SemaphoreType.DMA,
    ],
)


_RB = 1024


def _h2_body(z_ref, w_ref, degp_ref, h2_ref):
    deg = degp_ref[0, :] + degp_ref[1, :] + 1.0
    dis = lax.rsqrt(deg)
    h = jnp.dot(z_ref[...], w_ref[...], preferred_element_type=jnp.float32)
    h2_ref[...] = h * dis[:, None]


_h2_call = pl.pallas_call(
    _h2_body,
    grid=(NP // _RB,),
    in_specs=[
        pl.BlockSpec((_RB, D), lambda i: (i, 0)),
        pl.BlockSpec((D, D), lambda i: (0, 0)),
        pl.BlockSpec((NC, _RB), lambda i: (0, i)),
    ],
    out_specs=pl.BlockSpec((_RB, D), lambda i: (i, 0)),
    out_shape=jax.ShapeDtypeStruct((NP, D), jnp.float32),
)


def _g_body(acc_ref, h2_ref, degp_ref, b_ref, g_ref):
    deg = degp_ref[0, :] + degp_ref[1, :] + 1.0
    dis = lax.rsqrt(deg)
    t = (acc_ref[0] + acc_ref[1] + h2_ref[...]) * dis[:, None] + b_ref[...]
    g_ref[...] = jnp.maximum(t, 0.0)


_g_call = pl.pallas_call(
    _g_body,
    grid=(NP // _RB,),
    in_specs=[
        pl.BlockSpec((NC, _RB, D), lambda i: (0, i, 0)),
        pl.BlockSpec((_RB, D), lambda i: (i, 0)),
        pl.BlockSpec((NC, _RB), lambda i: (0, i)),
        pl.BlockSpec((1, D), lambda i: (0, 0)),
    ],
    out_specs=pl.BlockSpec((_RB, D), lambda i: (i, 0)),
    out_shape=jax.ShapeDtypeStruct((NP, D), jnp.float32),
)


_MB = 200


def _mm_body(gr_ref, gt_ref, out_ref):
    out_ref[...] = jnp.dot(gr_ref[...], gt_ref[...], preferred_element_type=jnp.float32)


_mm_call = pl.pallas_call(
    _mm_body,
    grid=(N_NODES // _MB,),
    in_specs=[
        pl.BlockSpec((_MB, D), lambda i: (i, 0)),
        pl.BlockSpec((D, N_NODES), lambda i: (0, 0)),
    ],
    out_specs=pl.BlockSpec((_MB, N_NODES), lambda i: (i, 0)),
    out_shape=jax.ShapeDtypeStruct((N_NODES, N_NODES), jnp.float32),
)


def kernel(z, edge_index, W, b):
    src = edge_index[0]
    dst = edge_index[1]
    pad = (N_NODES + (jnp.arange(EP - N_EDGES) % (NP - N_NODES))).astype(jnp.int32)
    srcp = jnp.concatenate([src, pad]).reshape(ROWS, 128)
    dstp = jnp.concatenate([dst, pad]).reshape(ROWS, 128)
    zp = jnp.zeros((NP, D), jnp.float32).at[:N_NODES].set(z)

    degp = _deg_call(dstp)
    h2 = _h2_call(zp, W, degp)
    accp = _agg_call(srcp, dstp, h2)
    g = _g_call(accp, h2, degp, b.reshape(1, D))
    g10 = g[:N_NODES]
    return _mm_call(g10, g10.T)


# speedup vs baseline: 24.7237x; 1.1703x over previous
import functools

import jax
import jax.numpy as jnp
from jax import lax
from jax.experimental import pallas as pl
from jax.experimental.pallas import tpu as pltpu
from jax.experimental.pallas import tpu_sc as plsc

NC, NS, L = 2, 16, 16
N_NODES = 10000
D = 128
N_EDGES = 320000
NP = 10240
EP = 327680
ROWS = EP // 128
RPW = ROWS // (NC * NS)
NPT = NP // NS

_sc_mesh = plsc.VectorSubcoreMesh(core_axis_name="c", subcore_axis_name="s")


def _deg_body(dst_hbm, deg_hbm, idx_v, ones_v, buf_v, deg_sh, sem):
    c = lax.axis_index("c")
    s = lax.axis_index("s")
    row0 = (c * NS + s) * RPW
    for i in range(128 // 16):
        ones_v[pl.ds(i * 16, 16)] = jnp.ones((16,), jnp.float32)
    for i in range(NPT // 16):
        buf_v[pl.ds(i * 16, 16)] = jnp.zeros((16,), jnp.float32)
    pltpu.sync_copy(buf_v, deg_sh.at[pl.ds(s * NPT, NPT)])
    pltpu.sync_copy(dst_hbm.at[pl.ds(row0, RPW)], idx_v)
    plsc.subcore_barrier()

    def step(j, carry):
        pltpu.sync_copy(ones_v, deg_sh.at[idx_v.at[j]], add=True)
        return carry

    lax.fori_loop(0, RPW, step, 0)
    plsc.subcore_barrier()
    pltpu.sync_copy(deg_sh.at[pl.ds(s * NPT, NPT)], buf_v)
    pltpu.sync_copy(buf_v, deg_hbm.at[c, pl.ds(s * NPT, NPT)])


_deg_call = pl.kernel(
    _deg_body,
    out_type=jax.ShapeDtypeStruct((NC, NP), jnp.float32),
    mesh=_sc_mesh,
    scratch_types=[
        pltpu.VMEM((RPW, 128), jnp.int32),
        pltpu.VMEM((128,), jnp.float32),
        pltpu.VMEM((NPT,), jnp.float32),
        pltpu.VMEM_SHARED((NP,), jnp.float32),
        pltpu.SemaphoreType.DMA,
    ],
)


_CH = 16
_NCH = RPW // _CH


def _agg_body(src_hbm, dst_hbm, h2_hbm, acc_hbm, sidx, didx, rows_a, rows_b,
              acc_sh, sem_a, sem_b):
    c = lax.axis_index("c")
    s = lax.axis_index("s")
    row0 = (c * NS + s) * RPW
    pltpu.sync_copy(h2_hbm.at[pl.ds(NP - 64, 64)], rows_b.at[pl.ds(0, 64)])
    for t in range(NPT // 64):
        pltpu.sync_copy(rows_b.at[pl.ds(0, 64)], acc_sh.at[pl.ds(s * NPT + t * 64, 64)])
    plsc.subcore_barrier()

    for ch in range(_NCH):
        pltpu.sync_copy(src_hbm.at[pl.ds(row0 + ch * _CH, _CH)], sidx)
        pltpu.sync_copy(dst_hbm.at[pl.ds(row0 + ch * _CH, _CH)], didx)
        pltpu.async_copy(h2_hbm.at[sidx.at[0]], rows_a, sem_a)

        def step(p, carry):
            i1 = 2 * p + 1
            pltpu.async_copy(h2_hbm.at[sidx.at[i1]], rows_b, sem_b)
            pltpu.make_async_copy(h2_hbm.at[sidx.at[0]], rows_a, sem_a).wait()
            pltpu.sync_copy(rows_a, acc_sh.at[didx.at[2 * p]], add=True)
            nxt = jnp.minimum(i1 + 1, _CH - 1)
            pltpu.async_copy(h2_hbm.at[sidx.at[nxt]], rows_a, sem_a)
            pltpu.make_async_copy(h2_hbm.at[sidx.at[0]], rows_b, sem_b).wait()
            pltpu.sync_copy(rows_b, acc_sh.at[didx.at[i1]], add=True)
            return carry

        lax.fori_loop(0, _CH // 2, step, 0)
        pltpu.make_async_copy(h2_hbm.at[sidx.at[0]], rows_a, sem_a).wait()
    plsc.subcore_barrier()
    for t in range(NPT // 128):
        pltpu.sync_copy(acc_sh.at[pl.ds(s * NPT + t * 128, 128)], rows_a)
        pltpu.sync_copy(rows_a, acc_hbm.at[c, pl.ds(s * NPT + t * 128, 128)])


_agg_call = pl.kernel(
    _agg_body,
    out_type=jax.ShapeDtypeStruct((NC, NP, D), jnp.float32),
    mesh=_sc_mesh,
    scratch_types=[
        pltpu.VMEM((_CH, 128), jnp.int32),
        pltpu.VMEM((_CH, 128), jnp.int32),
        pltpu.VMEM((128, D), jnp.float32),
        pltpu.VMEM((128, D), jnp.float32),
        pltpu.VMEM_SHARED((NP, D), jnp.float32),
        pltpu.SemaphoreType.DMA,
        pltpu.SemaphoreType.DMA,
    ],
)


_RB = 1024


def _h2_body(z_ref, w_ref, degp_ref, h2_ref):
    deg = degp_ref[0, :] + degp_ref[1, :] + 1.0
    dis = lax.rsqrt(deg)
    h = jnp.dot(z_ref[...], w_ref[...], preferred_element_type=jnp.float32)
    h2_ref[...] = h * dis[:, None]


_h2_call = pl.pallas_call(
    _h2_body,
    grid=(NP // _RB,),
    in_specs=[
        pl.BlockSpec((_RB, D), lambda i: (i, 0)),
        pl.BlockSpec((D, D), lambda i: (0, 0)),
        pl.BlockSpec((NC, _RB), lambda i: (0, i)),
    ],
    out_specs=pl.BlockSpec((_RB, D), lambda i: (i, 0)),
    out_shape=jax.ShapeDtypeStruct((NP, D), jnp.float32),
)


def _g_body(acc_ref, h2_ref, degp_ref, b_ref, g_ref):
    deg = degp_ref[0, :] + degp_ref[1, :] + 1.0
    dis = lax.rsqrt(deg)
    t = (acc_ref[0] + acc_ref[1] + h2_ref[...]) * dis[:, None] + b_ref[...]
    g_ref[...] = jnp.maximum(t, 0.0)


_g_call = pl.pallas_call(
    _g_body,
    grid=(NP // _RB,),
    in_specs=[
        pl.BlockSpec((NC, _RB, D), lambda i: (0, i, 0)),
        pl.BlockSpec((_RB, D), lambda i: (i, 0)),
        pl.BlockSpec((NC, _RB), lambda i: (0, i)),
        pl.BlockSpec((1, D), lambda i: (0, 0)),
    ],
    out_specs=pl.BlockSpec((_RB, D), lambda i: (i, 0)),
    out_shape=jax.ShapeDtypeStruct((NP, D), jnp.float32),
)


_MB = 200


def _mm_body(gr_ref, gt_ref, out_ref):
    out_ref[...] = jnp.dot(gr_ref[...], gt_ref[...], preferred_element_type=jnp.float32)


_mm_call = pl.pallas_call(
    _mm_body,
    grid=(N_NODES // _MB,),
    in_specs=[
        pl.BlockSpec((_MB, D), lambda i: (i, 0)),
        pl.BlockSpec((D, N_NODES), lambda i: (0, 0)),
    ],
    out_specs=pl.BlockSpec((_MB, N_NODES), lambda i: (i, 0)),
    out_shape=jax.ShapeDtypeStruct((N_NODES, N_NODES), jnp.float32),
)


def kernel(z, edge_index, W, b):
    src = edge_index[0]
    dst = edge_index[1]
    pad = (N_NODES + (jnp.arange(EP - N_EDGES) % (NP - N_NODES))).astype(jnp.int32)
    srcp = jnp.concatenate([src, pad]).reshape(ROWS, 128)
    dstp = jnp.concatenate([dst, pad]).reshape(ROWS, 128)
    zp = jnp.zeros((NP, D), jnp.float32).at[:N_NODES].set(z)

    degp = _deg_call(dstp)
    h2 = _h2_call(zp, W, degp)
    accp = _agg_call(srcp, dstp, h2)
    g = _g_call(accp, h2, degp, b.reshape(1, D))
    g10 = g[:N_NODES]
    return _mm_call(g10, g10.T)


# speedup vs baseline: 24.9819x; 1.0104x over previous
import functools

import jax
import jax.numpy as jnp
from jax import lax
from jax.experimental import pallas as pl
from jax.experimental.pallas import tpu as pltpu
from jax.experimental.pallas import tpu_sc as plsc

NC, NS, L = 2, 16, 16
N_NODES = 10000
D = 128
N_EDGES = 320000
NP = 10240
EP = 327680
ROWS = EP // 128
RPW = ROWS // (NC * NS)
NPT = NP // NS

_sc_mesh = plsc.VectorSubcoreMesh(core_axis_name="c", subcore_axis_name="s")


def _deg_body(dst_hbm, deg_hbm, idx_v, ones_v, buf_v, deg_sh, sem):
    c = lax.axis_index("c")
    s = lax.axis_index("s")
    row0 = (c * NS + s) * RPW
    for i in range(128 // 16):
        ones_v[pl.ds(i * 16, 16)] = jnp.ones((16,), jnp.float32)
    for i in range(NPT // 16):
        buf_v[pl.ds(i * 16, 16)] = jnp.zeros((16,), jnp.float32)
    pltpu.sync_copy(buf_v, deg_sh.at[pl.ds(s * NPT, NPT)])
    pltpu.sync_copy(dst_hbm.at[pl.ds(row0, RPW)], idx_v)
    plsc.subcore_barrier()

    def step(j, carry):
        pltpu.sync_copy(ones_v, deg_sh.at[idx_v.at[j]], add=True)
        return carry

    lax.fori_loop(0, RPW, step, 0)
    plsc.subcore_barrier()
    pltpu.sync_copy(deg_sh.at[pl.ds(s * NPT, NPT)], buf_v)
    pltpu.sync_copy(buf_v, deg_hbm.at[c, pl.ds(s * NPT, NPT)])


_deg_call = pl.kernel(
    _deg_body,
    out_type=jax.ShapeDtypeStruct((NC, NP), jnp.float32),
    mesh=_sc_mesh,
    scratch_types=[
        pltpu.VMEM((RPW, 128), jnp.int32),
        pltpu.VMEM((128,), jnp.float32),
        pltpu.VMEM((NPT,), jnp.float32),
        pltpu.VMEM_SHARED((NP,), jnp.float32),
        pltpu.SemaphoreType.DMA,
    ],
)


_CH = 16
_NCH = RPW // _CH


def _agg_body(src_hbm, dst_hbm, h2_hbm, acc_hbm, sidx, didx, rows_a, rows_b,
              acc_sh, sem_a, sem_b):
    c = lax.axis_index("c")
    s = lax.axis_index("s")
    row0 = (c * NS + s) * RPW
    pltpu.sync_copy(h2_hbm.at[pl.ds(NP - 64, 64)], rows_b.at[pl.ds(0, 64)])
    for t in range(NPT // 64):
        pltpu.sync_copy(rows_b.at[pl.ds(0, 64)], acc_sh.at[pl.ds(s * NPT + t * 64, 64)])
    plsc.subcore_barrier()

    for ch in range(_NCH):
        pltpu.sync_copy(src_hbm.at[pl.ds(row0 + ch * _CH, _CH)], sidx)
        pltpu.sync_copy(dst_hbm.at[pl.ds(row0 + ch * _CH, _CH)], didx)
        pltpu.async_copy(h2_hbm.at[sidx.at[0]], rows_a, sem_a)

        def step(p, carry):
            i1 = 2 * p + 1
            pltpu.async_copy(h2_hbm.at[sidx.at[i1]], rows_b, sem_b)
            pltpu.make_async_copy(h2_hbm.at[sidx.at[0]], rows_a, sem_a).wait()
            pltpu.sync_copy(rows_a, acc_sh.at[didx.at[2 * p]], add=True)
            nxt = jnp.minimum(i1 + 1, _CH - 1)
            pltpu.async_copy(h2_hbm.at[sidx.at[nxt]], rows_a, sem_a)
            pltpu.make_async_copy(h2_hbm.at[sidx.at[0]], rows_b, sem_b).wait()
            pltpu.sync_copy(rows_b, acc_sh.at[didx.at[i1]], add=True)
            return carry

        lax.fori_loop(0, _CH // 2, step, 0)
        pltpu.make_async_copy(h2_hbm.at[sidx.at[0]], rows_a, sem_a).wait()
    plsc.subcore_barrier()
    for t in range(NPT // 128):
        pltpu.sync_copy(acc_sh.at[pl.ds(s * NPT + t * 128, 128)], rows_a)
        pltpu.sync_copy(rows_a, acc_hbm.at[c, pl.ds(s * NPT + t * 128, 128)])


_agg_call = pl.kernel(
    _agg_body,
    out_type=jax.ShapeDtypeStruct((NC, NP, D), jnp.float32),
    mesh=_sc_mesh,
    scratch_types=[
        pltpu.VMEM((_CH, 128), jnp.int32),
        pltpu.VMEM((_CH, 128), jnp.int32),
        pltpu.VMEM((128, D), jnp.float32),
        pltpu.VMEM((128, D), jnp.float32),
        pltpu.VMEM_SHARED((NP, D), jnp.float32),
        pltpu.SemaphoreType.DMA,
        pltpu.SemaphoreType.DMA,
    ],
)


_RB = 1024


def _zw_body(z_ref, w_ref, h_ref):
    h_ref[...] = jnp.dot(z_ref[...], w_ref[...], preferred_element_type=jnp.float32)


_zw_call = pl.pallas_call(
    _zw_body,
    grid=(NP // _RB,),
    in_specs=[
        pl.BlockSpec((_RB, D), lambda i: (i, 0)),
        pl.BlockSpec((D, D), lambda i: (0, 0)),
    ],
    out_specs=pl.BlockSpec((_RB, D), lambda i: (i, 0)),
    out_shape=jax.ShapeDtypeStruct((NP, D), jnp.float32),
)


def _scale_body(h_ref, degp_ref, h2_ref):
    deg = degp_ref[0, :] + degp_ref[1, :] + 1.0
    dis = lax.rsqrt(deg)
    h2_ref[...] = h_ref[...] * dis[:, None]


_scale_call = pl.pallas_call(
    _scale_body,
    grid=(NP // _RB,),
    in_specs=[
        pl.BlockSpec((_RB, D), lambda i: (i, 0)),
        pl.BlockSpec((NC, _RB), lambda i: (0, i)),
    ],
    out_specs=pl.BlockSpec((_RB, D), lambda i: (i, 0)),
    out_shape=jax.ShapeDtypeStruct((NP, D), jnp.float32),
)


def _g_body(acc_ref, h2_ref, degp_ref, b_ref, g_ref):
    deg = degp_ref[0, :] + degp_ref[1, :] + 1.0
    dis = lax.rsqrt(deg)
    t = (acc_ref[0] + acc_ref[1] + h2_ref[...]) * dis[:, None] + b_ref[...]
    g_ref[...] = jnp.maximum(t, 0.0).astype(jnp.bfloat16)


_g_call = pl.pallas_call(
    _g_body,
    grid=(NP // _RB,),
    in_specs=[
        pl.BlockSpec((NC, _RB, D), lambda i: (0, i, 0)),
        pl.BlockSpec((_RB, D), lambda i: (i, 0)),
        pl.BlockSpec((NC, _RB), lambda i: (0, i)),
        pl.BlockSpec((1, D), lambda i: (0, 0)),
    ],
    out_specs=pl.BlockSpec((_RB, D), lambda i: (i, 0)),
    out_shape=jax.ShapeDtypeStruct((NP, D), jnp.bfloat16),
)


_MB = 200


def _mm_body(gr_ref, gt_ref, out_ref):
    out_ref[...] = jnp.dot(gr_ref[...], gt_ref[...], preferred_element_type=jnp.float32)


_mm_call = pl.pallas_call(
    _mm_body,
    grid=(N_NODES // _MB,),
    in_specs=[
        pl.BlockSpec((_MB, D), lambda i: (i, 0)),
        pl.BlockSpec((D, N_NODES), lambda i: (0, 0)),
    ],
    out_specs=pl.BlockSpec((_MB, N_NODES), lambda i: (i, 0)),
    out_shape=jax.ShapeDtypeStruct((N_NODES, N_NODES), jnp.float32),
)


def kernel(z, edge_index, W, b):
    src = edge_index[0]
    dst = edge_index[1]
    pad = (N_NODES + (jnp.arange(EP - N_EDGES) % (NP - N_NODES))).astype(jnp.int32)
    srcp = jnp.concatenate([src, pad]).reshape(ROWS, 128)
    dstp = jnp.concatenate([dst, pad]).reshape(ROWS, 128)
    zp = jnp.zeros((NP, D), jnp.float32).at[:N_NODES].set(z)

    h = _zw_call(zp, W)
    degp = _deg_call(dstp)
    h2 = _scale_call(h, degp)
    accp = _agg_call(srcp, dstp, h2)
    g = _g_call(accp, h2, degp, b.reshape(1, D))
    g10 = g[:N_NODES]
    return _mm_call(g10, g10.T)


# speedup vs baseline: 25.4121x; 1.0172x over previous
import functools

import jax
import jax.numpy as jnp
from jax import lax
from jax.experimental import pallas as pl
from jax.experimental.pallas import tpu as pltpu
from jax.experimental.pallas import tpu_sc as plsc

NC, NS, L = 2, 16, 16
N_NODES = 10000
D = 128
N_EDGES = 320000
NP = 10240
EP = 327680
NPT = NP // NS
_BW = 64
_BROWS = EP // _BW
_BPW = _BROWS // (NC * NS)
_CH = 32
_NCH = _BPW // _CH
_RING = 4

_sc_mesh = plsc.VectorSubcoreMesh(core_axis_name="c", subcore_axis_name="s")


def _deg_body(dst_hbm, deg_hbm, idx_v, ones_v, buf_v, deg_sh, sem):
    c = lax.axis_index("c")
    s = lax.axis_index("s")
    row0 = (c * NS + s) * _BPW
    for i in range(_BW // 16):
        ones_v[pl.ds(i * 16, 16)] = jnp.ones((16,), jnp.float32)
    for i in range(NPT // 16):
        buf_v[pl.ds(i * 16, 16)] = jnp.zeros((16,), jnp.float32)
    pltpu.sync_copy(buf_v, deg_sh.at[pl.ds(s * NPT, NPT)])
    pltpu.sync_copy(dst_hbm.at[pl.ds(row0, _BPW)], idx_v)
    plsc.subcore_barrier()

    def step(j, carry):
        pltpu.sync_copy(ones_v, deg_sh.at[idx_v.at[j]], add=True)
        return carry

    lax.fori_loop(0, _BPW, step, 0)
    plsc.subcore_barrier()
    pltpu.sync_copy(deg_sh.at[pl.ds(s * NPT, NPT)], buf_v)
    pltpu.sync_copy(buf_v, deg_hbm.at[c, pl.ds(s * NPT, NPT)])


_deg_call = pl.kernel(
    _deg_body,
    out_type=jax.ShapeDtypeStruct((NC, NP), jnp.float32),
    mesh=_sc_mesh,
    scratch_types=[
        pltpu.VMEM((_BPW, _BW), jnp.int32),
        pltpu.VMEM((_BW,), jnp.float32),
        pltpu.VMEM((NPT,), jnp.float32),
        pltpu.VMEM_SHARED((NP,), jnp.float32),
        pltpu.SemaphoreType.DMA,
    ],
)


def _agg_body(src_hbm, dst_hbm, h2_hbm, acc_hbm, sidx, didx,
              r0, r1, r2, r3, acc_sh, s0, s1, s2, s3):
    c = lax.axis_index("c")
    s = lax.axis_index("s")
    rows = [r0, r1, r2, r3]
    sems = [s0, s1, s2, s3]
    row0 = (c * NS + s) * _BPW
    pltpu.sync_copy(h2_hbm.at[pl.ds(NP - 64, 64)], r3)
    for t in range(NPT // 64):
        pltpu.sync_copy(r3, acc_sh.at[pl.ds(s * NPT + t * 64, 64)])
    plsc.subcore_barrier()

    for ch in range(_NCH):
        pltpu.sync_copy(src_hbm.at[pl.ds(row0 + ch * _CH, _CH)], sidx)
        pltpu.sync_copy(dst_hbm.at[pl.ds(row0 + ch * _CH, _CH)], didx)
        for q in range(_RING - 1):
            pltpu.async_copy(h2_hbm.at[sidx.at[q]], rows[q], sems[q])

        def grp(p, carry):
            for q in range(_RING):
                i = _RING * p + q
                nxt = jnp.minimum(i + _RING - 1, _CH - 1)
                pltpu.async_copy(
                    h2_hbm.at[sidx.at[nxt]], rows[(q + 3) % 4], sems[(q + 3) % 4])
                pltpu.make_async_copy(h2_hbm.at[sidx.at[0]], rows[q], sems[q]).wait()
                pltpu.sync_copy(rows[q], acc_sh.at[didx.at[i]], add=True)
            return carry

        lax.fori_loop(0, _CH // _RING, grp, 0)
        for q in range(_RING - 1):
            pltpu.make_async_copy(h2_hbm.at[sidx.at[0]], rows[q], sems[q]).wait()
    plsc.subcore_barrier()
    for t in range(NPT // 64):
        pltpu.sync_copy(acc_sh.at[pl.ds(s * NPT + t * 64, 64)], r0)
        pltpu.sync_copy(r0, acc_hbm.at[c, pl.ds(s * NPT + t * 64, 64)])


_agg_call = pl.kernel(
    _agg_body,
    out_type=jax.ShapeDtypeStruct((NC, NP, D), jnp.float32),
    mesh=_sc_mesh,
    scratch_types=[
        pltpu.VMEM((_CH, _BW), jnp.int32),
        pltpu.VMEM((_CH, _BW), jnp.int32),
        pltpu.VMEM((_BW, D), jnp.float32),
        pltpu.VMEM((_BW, D), jnp.float32),
        pltpu.VMEM((_BW, D), jnp.float32),
        pltpu.VMEM((_BW, D), jnp.float32),
        pltpu.VMEM_SHARED((NP, D), jnp.float32),
        pltpu.SemaphoreType.DMA,
        pltpu.SemaphoreType.DMA,
        pltpu.SemaphoreType.DMA,
        pltpu.SemaphoreType.DMA,
    ],
)


_RB = 1024


def _zw_body(z_ref, w_ref, h_ref):
    h_ref[...] = jnp.dot(z_ref[...], w_ref[...], preferred_element_type=jnp.float32)


_zw_call = pl.pallas_call(
    _zw_body,
    grid=(NP // _RB,),
    in_specs=[
        pl.BlockSpec((_RB, D), lambda i: (i, 0)),
        pl.BlockSpec((D, D), lambda i: (0, 0)),
    ],
    out_specs=pl.BlockSpec((_RB, D), lambda i: (i, 0)),
    out_shape=jax.ShapeDtypeStruct((NP, D), jnp.float32),
)


def _scale_body(h_ref, degp_ref, h2_ref):
    deg = degp_ref[0, :] + degp_ref[1, :] + 1.0
    dis = lax.rsqrt(deg)
    h2_ref[...] = h_ref[...] * dis[:, None]


_scale_call = pl.pallas_call(
    _scale_body,
    grid=(NP // _RB,),
    in_specs=[
        pl.BlockSpec((_RB, D), lambda i: (i, 0)),
        pl.BlockSpec((NC, _RB), lambda i: (0, i)),
    ],
    out_specs=pl.BlockSpec((_RB, D), lambda i: (i, 0)),
    out_shape=jax.ShapeDtypeStruct((NP, D), jnp.float32),
)


def _g_body(acc_ref, h2_ref, degp_ref, b_ref, g_ref):
    deg = degp_ref[0, :] + degp_ref[1, :] + 1.0
    dis = lax.rsqrt(deg)
    t = (acc_ref[0] + acc_ref[1] + h2_ref[...]) * dis[:, None] + b_ref[...]
    g_ref[...] = jnp.maximum(t, 0.0).astype(jnp.bfloat16)


_g_call = pl.pallas_call(
    _g_body,
    grid=(NP // _RB,),
    in_specs=[
        pl.BlockSpec((NC, _RB, D), lambda i: (0, i, 0)),
        pl.BlockSpec((_RB, D), lambda i: (i, 0)),
        pl.BlockSpec((NC, _RB), lambda i: (0, i)),
        pl.BlockSpec((1, D), lambda i: (0, 0)),
    ],
    out_specs=pl.BlockSpec((_RB, D), lambda i: (i, 0)),
    out_shape=jax.ShapeDtypeStruct((NP, D), jnp.bfloat16),
)


_MB = 200


def _mm_body(gr_ref, gt_ref, out_ref):
    out_ref[...] = jnp.dot(gr_ref[...], gt_ref[...], preferred_element_type=jnp.float32)


_mm_call = pl.pallas_call(
    _mm_body,
    grid=(N_NODES // _MB,),
    in_specs=[
        pl.BlockSpec((_MB, D), lambda i: (i, 0)),
        pl.BlockSpec((D, N_NODES), lambda i: (0, 0)),
    ],
    out_specs=pl.BlockSpec((_MB, N_NODES), lambda i: (i, 0)),
    out_shape=jax.ShapeDtypeStruct((N_NODES, N_NODES), jnp.float32),
)


def kernel(z, edge_index, W, b):
    src = edge_index[0]
    dst = edge_index[1]
    pad = (N_NODES + (jnp.arange(EP - N_EDGES) % (NP - N_NODES))).astype(jnp.int32)
    srcp = jnp.concatenate([src, pad]).reshape(_BROWS, _BW)
    dstp = jnp.concatenate([dst, pad]).reshape(_BROWS, _BW)
    zp = jnp.zeros((NP, D), jnp.float32).at[:N_NODES].set(z)

    h = _zw_call(zp, W)
    degp = _deg_call(dstp)
    h2 = _scale_call(h, degp)
    accp = _agg_call(srcp, dstp, h2)
    g = _g_call(accp, h2, degp, b.reshape(1, D))
    g10 = g[:N_NODES]
    return _mm_call(g10, g10.T)


# speedup vs baseline: 25.6737x; 1.0103x over previous
import functools

import jax
import jax.numpy as jnp
from jax import lax
from jax.experimental import pallas as pl
from jax.experimental.pallas import tpu as pltpu
from jax.experimental.pallas import tpu_sc as plsc

NC, NS, L = 2, 16, 16
N_NODES = 10000
D = 128
N_EDGES = 320000
NP = 10240
EP = 327680
NPT = NP // NS
_BW = 64
_BROWS = EP // _BW
_BPW = _BROWS // (NC * NS)
_CH = 32
_NCH = _BPW // _CH
_RING = 4

_sc_mesh = plsc.VectorSubcoreMesh(core_axis_name="c", subcore_axis_name="s")


def _deg_body(dst_hbm, deg_hbm, idx_v, ones_v, buf_v, deg_sh, sem):
    c = lax.axis_index("c")
    s = lax.axis_index("s")
    row0 = (c * NS + s) * _BPW
    for i in range(_BW // 16):
        ones_v[pl.ds(i * 16, 16)] = jnp.ones((16,), jnp.float32)
    for i in range(NPT // 16):
        buf_v[pl.ds(i * 16, 16)] = jnp.zeros((16,), jnp.float32)
    pltpu.sync_copy(buf_v, deg_sh.at[pl.ds(s * NPT, NPT)])
    pltpu.sync_copy(dst_hbm.at[pl.ds(row0, _BPW)], idx_v)
    plsc.subcore_barrier()

    def step(j, carry):
        pltpu.sync_copy(ones_v, deg_sh.at[idx_v.at[j]], add=True)
        return carry

    lax.fori_loop(0, _BPW, step, 0)
    plsc.subcore_barrier()
    pltpu.sync_copy(deg_sh.at[pl.ds(s * NPT, NPT)], buf_v)
    pltpu.sync_copy(buf_v, deg_hbm.at[c, pl.ds(s * NPT, NPT)])


_deg_call = pl.kernel(
    _deg_body,
    out_type=jax.ShapeDtypeStruct((NC, NP), jnp.float32),
    mesh=_sc_mesh,
    scratch_types=[
        pltpu.VMEM((_BPW, _BW), jnp.int32),
        pltpu.VMEM((_BW,), jnp.float32),
        pltpu.VMEM((NPT,), jnp.float32),
        pltpu.VMEM_SHARED((NP,), jnp.float32),
        pltpu.SemaphoreType.DMA,
    ],
)


def _agg_body(src_hbm, dst_hbm, h2_hbm, acc_hbm, sidx, didx,
              r0, r1, r2, r3, acc_sh, s0, s1, s2, s3):
    c = lax.axis_index("c")
    s = lax.axis_index("s")
    rows = [r0, r1, r2, r3]
    sems = [s0, s1, s2, s3]
    row0 = (c * NS + s) * _BPW
    pltpu.sync_copy(h2_hbm.at[pl.ds(NP - 64, 64)], r3)
    for t in range(NPT // 64):
        pltpu.sync_copy(r3, acc_sh.at[pl.ds(s * NPT + t * 64, 64)])
    plsc.subcore_barrier()

    for ch in range(_NCH):
        pltpu.sync_copy(src_hbm.at[pl.ds(row0 + ch * _CH, _CH)], sidx)
        pltpu.sync_copy(dst_hbm.at[pl.ds(row0 + ch * _CH, _CH)], didx)
        for q in range(_RING - 1):
            pltpu.async_copy(h2_hbm.at[sidx.at[q]], rows[q], sems[q])

        def grp(p, carry):
            for q in range(_RING):
                i = _RING * p + q
                nxt = jnp.minimum(i + _RING - 1, _CH - 1)
                pltpu.async_copy(
                    h2_hbm.at[sidx.at[nxt]], rows[(q + 3) % 4], sems[(q + 3) % 4])
                pltpu.make_async_copy(h2_hbm.at[sidx.at[0]], rows[q], sems[q]).wait()
                pltpu.sync_copy(rows[q], acc_sh.at[didx.at[i]], add=True)
            return carry

        lax.fori_loop(0, _CH // _RING, grp, 0)
        for q in range(_RING - 1):
            pltpu.make_async_copy(h2_hbm.at[sidx.at[0]], rows[q], sems[q]).wait()
    plsc.subcore_barrier()
    for t in range(NPT // 64):
        pltpu.sync_copy(acc_sh.at[pl.ds(s * NPT + t * 64, 64)], r0)
        pltpu.sync_copy(r0, acc_hbm.at[c, pl.ds(s * NPT + t * 64, 64)])


_agg_call = pl.kernel(
    _agg_body,
    out_type=jax.ShapeDtypeStruct((NC, NP, D), jnp.float32),
    mesh=_sc_mesh,
    scratch_types=[
        pltpu.VMEM((_CH, _BW), jnp.int32),
        pltpu.VMEM((_CH, _BW), jnp.int32),
        pltpu.VMEM((_BW, D), jnp.float32),
        pltpu.VMEM((_BW, D), jnp.float32),
        pltpu.VMEM((_BW, D), jnp.float32),
        pltpu.VMEM((_BW, D), jnp.float32),
        pltpu.VMEM_SHARED((NP, D), jnp.float32),
        pltpu.SemaphoreType.DMA,
        pltpu.SemaphoreType.DMA,
        pltpu.SemaphoreType.DMA,
        pltpu.SemaphoreType.DMA,
    ],
)


_RB = 1024


def _zw_body(z_ref, w_ref, h_ref):
    h_ref[...] = jnp.dot(z_ref[...], w_ref[...], preferred_element_type=jnp.float32)


_zw_call = pl.pallas_call(
    _zw_body,
    grid=(NP // _RB,),
    in_specs=[
        pl.BlockSpec((_RB, D), lambda i: (i, 0)),
        pl.BlockSpec((D, D), lambda i: (0, 0)),
    ],
    out_specs=pl.BlockSpec((_RB, D), lambda i: (i, 0)),
    out_shape=jax.ShapeDtypeStruct((NP, D), jnp.float32),
)


def _scale_body(h_ref, degp_ref, h2_ref):
    deg = degp_ref[0, :] + degp_ref[1, :] + 1.0
    dis = lax.rsqrt(deg)
    h2_ref[...] = h_ref[...] * dis[:, None]


_scale_call = pl.pallas_call(
    _scale_body,
    grid=(NP // _RB,),
    in_specs=[
        pl.BlockSpec((_RB, D), lambda i: (i, 0)),
        pl.BlockSpec((NC, _RB), lambda i: (0, i)),
    ],
    out_specs=pl.BlockSpec((_RB, D), lambda i: (i, 0)),
    out_shape=jax.ShapeDtypeStruct((NP, D), jnp.float32),
)


def _g_body(acc_ref, h2_ref, degp_ref, b_ref, g_ref, gt_ref):
    deg = degp_ref[0, :] + degp_ref[1, :] + 1.0
    dis = lax.rsqrt(deg)
    t = (acc_ref[0] + acc_ref[1] + h2_ref[...]) * dis[:, None] + b_ref[...]
    gb = jnp.maximum(t, 0.0).astype(jnp.bfloat16)
    g_ref[...] = gb
    gt_ref[...] = gb.T


_g_call = pl.pallas_call(
    _g_body,
    grid=(NP // _RB,),
    in_specs=[
        pl.BlockSpec((NC, _RB, D), lambda i: (0, i, 0)),
        pl.BlockSpec((_RB, D), lambda i: (i, 0)),
        pl.BlockSpec((NC, _RB), lambda i: (0, i)),
        pl.BlockSpec((1, D), lambda i: (0, 0)),
    ],
    out_specs=[
        pl.BlockSpec((_RB, D), lambda i: (i, 0)),
        pl.BlockSpec((D, _RB), lambda i: (0, i)),
    ],
    out_shape=[
        jax.ShapeDtypeStruct((NP, D), jnp.bfloat16),
        jax.ShapeDtypeStruct((128, N_NODES), jnp.bfloat16),
    ],
)


_MB = 400


def _mm_body(gr_ref, gt_ref, out_ref):
    out_ref[...] = jnp.dot(gr_ref[...], gt_ref[...], preferred_element_type=jnp.float32)


_mm_call = pl.pallas_call(
    _mm_body,
    grid=(N_NODES // _MB,),
    in_specs=[
        pl.BlockSpec((_MB, D), lambda i: (i, 0)),
        pl.BlockSpec((D, N_NODES), lambda i: (0, 0)),
    ],
    out_specs=pl.BlockSpec((_MB, N_NODES), lambda i: (i, 0)),
    out_shape=jax.ShapeDtypeStruct((N_NODES, N_NODES), jnp.float32),
)


def kernel(z, edge_index, W, b):
    src = edge_index[0]
    dst = edge_index[1]
    pad = (N_NODES + (jnp.arange(EP - N_EDGES) % (NP - N_NODES))).astype(jnp.int32)
    srcp = jnp.concatenate([src, pad]).reshape(_BROWS, _BW)
    dstp = jnp.concatenate([dst, pad]).reshape(_BROWS, _BW)
    zp = jnp.zeros((NP, D), jnp.float32).at[:N_NODES].set(z)

    h = _zw_call(zp, W)
    degp = _deg_call(dstp)
    h2 = _scale_call(h, degp)
    accp = _agg_call(srcp, dstp, h2)
    g, gt = _g_call(accp, h2, degp, b.reshape(1, D))
    return _mm_call(g, gt)


# speedup vs baseline: 25.7553x; 1.0032x over previous
import functools

import jax
import jax.numpy as jnp
from jax import lax
from jax.experimental import pallas as pl
from jax.experimental.pallas import tpu as pltpu
from jax.experimental.pallas import tpu_sc as plsc

NC, NS, L = 2, 16, 16
N_NODES = 10000
D = 128
N_EDGES = 320000
NP = 10240
EP = 327680
NPT = NP // NS
DROWS = EP // 128
DRPW = DROWS // (NC * NS)
_BW = 64
_BROWS = EP // _BW
_BPW = _BROWS // (NC * NS)
_CH = 32
_NCH = _BPW // _CH
_RING = 4

_sc_mesh = plsc.VectorSubcoreMesh(core_axis_name="c", subcore_axis_name="s")


def _deg_body(ei_hbm, deg_hbm, idx_v, ones_v, buf_v, deg_sh, sem):
    c = lax.axis_index("c")
    s = lax.axis_index("s")
    row0 = (c * NS + s) * DRPW
    for i in range(128 // 16):
        ones_v[pl.ds(i * 16, 16)] = jnp.ones((16,), jnp.float32)
    for i in range(NPT // 16):
        buf_v[pl.ds(i * 16, 16)] = jnp.zeros((16,), jnp.float32)
    pltpu.sync_copy(buf_v, deg_sh.at[pl.ds(s * NPT, NPT)])
    pltpu.sync_copy(ei_hbm.at[1, pl.ds(row0, DRPW)], idx_v)
    plsc.subcore_barrier()

    def step(j, carry):
        pltpu.sync_copy(ones_v, deg_sh.at[idx_v.at[j]], add=True)
        return carry

    lax.fori_loop(0, DRPW, step, 0)
    plsc.subcore_barrier()
    pltpu.sync_copy(deg_sh.at[pl.ds(s * NPT, NPT)], buf_v)
    pltpu.sync_copy(buf_v, deg_hbm.at[c, pl.ds(s * NPT, NPT)])


_deg_call = pl.kernel(
    _deg_body,
    out_type=jax.ShapeDtypeStruct((NC, NP), jnp.float32),
    mesh=_sc_mesh,
    scratch_types=[
        pltpu.VMEM((DRPW, 128), jnp.int32),
        pltpu.VMEM((128,), jnp.float32),
        pltpu.VMEM((NPT,), jnp.float32),
        pltpu.VMEM_SHARED((NP,), jnp.float32),
        pltpu.SemaphoreType.DMA,
    ],
)


def _agg_body(ei_hbm, h2_hbm, acc_hbm, sidx, didx,
              r0, r1, r2, r3, acc_sh, s0, s1, s2, s3):
    c = lax.axis_index("c")
    s = lax.axis_index("s")
    rows = [r0, r1, r2, r3]
    sems = [s0, s1, s2, s3]
    row0 = (c * NS + s) * _BPW
    pltpu.sync_copy(h2_hbm.at[pl.ds(NP - 64, 64)], r3)
    for t in range(NPT // 64):
        pltpu.sync_copy(r3, acc_sh.at[pl.ds(s * NPT + t * 64, 64)])
    plsc.subcore_barrier()

    for ch in range(_NCH):
        pltpu.sync_copy(ei_hbm.at[0, pl.ds(row0 + ch * _CH, _CH)], sidx)
        pltpu.sync_copy(ei_hbm.at[1, pl.ds(row0 + ch * _CH, _CH)], didx)
        for q in range(_RING - 1):
            pltpu.async_copy(h2_hbm.at[sidx.at[q]], rows[q], sems[q])

        def grp(p, carry):
            for q in range(_RING):
                i = _RING * p + q
                nxt = jnp.minimum(i + _RING - 1, _CH - 1)
                pltpu.async_copy(
                    h2_hbm.at[sidx.at[nxt]], rows[(q + 3) % 4], sems[(q + 3) % 4])
                pltpu.make_async_copy(h2_hbm.at[sidx.at[0]], rows[q], sems[q]).wait()
                pltpu.sync_copy(rows[q], acc_sh.at[didx.at[i]], add=True)
            return carry

        lax.fori_loop(0, _CH // _RING, grp, 0)
        for q in range(_RING - 1):
            pltpu.make_async_copy(h2_hbm.at[sidx.at[0]], rows[q], sems[q]).wait()
    plsc.subcore_barrier()
    for t in range(NPT // 64):
        pltpu.sync_copy(acc_sh.at[pl.ds(s * NPT + t * 64, 64)], r0)
        pltpu.sync_copy(r0, acc_hbm.at[c, pl.ds(s * NPT + t * 64, 64)])


_agg_call = pl.kernel(
    _agg_body,
    out_type=jax.ShapeDtypeStruct((NC, NP, D), jnp.float32),
    mesh=_sc_mesh,
    scratch_types=[
        pltpu.VMEM((_CH, _BW), jnp.int32),
        pltpu.VMEM((_CH, _BW), jnp.int32),
        pltpu.VMEM((_BW, D), jnp.float32),
        pltpu.VMEM((_BW, D), jnp.float32),
        pltpu.VMEM((_BW, D), jnp.float32),
        pltpu.VMEM((_BW, D), jnp.float32),
        pltpu.VMEM_SHARED((NP, D), jnp.float32),
        pltpu.SemaphoreType.DMA,
        pltpu.SemaphoreType.DMA,
        pltpu.SemaphoreType.DMA,
        pltpu.SemaphoreType.DMA,
    ],
)


_RB = 1024


def _zw_body(z_ref, w_ref, h_ref):
    h_ref[...] = jnp.dot(z_ref[...], w_ref[...], preferred_element_type=jnp.float32)


_zw_call = pl.pallas_call(
    _zw_body,
    grid=(NP // _RB,),
    in_specs=[
        pl.BlockSpec((_RB, D), lambda i: (i, 0)),
        pl.BlockSpec((D, D), lambda i: (0, 0)),
    ],
    out_specs=pl.BlockSpec((_RB, D), lambda i: (i, 0)),
    out_shape=jax.ShapeDtypeStruct((NP, D), jnp.float32),
)


def _scale_body(h_ref, degp_ref, h2_ref):
    deg = degp_ref[0, :] + degp_ref[1, :] + 1.0
    dis = lax.rsqrt(deg)
    h2_ref[...] = h_ref[...] * dis[:, None]


_RB2 = 2048

_scale_call = pl.pallas_call(
    _scale_body,
    grid=(NP // _RB2,),
    in_specs=[
        pl.BlockSpec((_RB2, D), lambda i: (i, 0)),
        pl.BlockSpec((NC, _RB2), lambda i: (0, i)),
    ],
    out_specs=pl.BlockSpec((_RB2, D), lambda i: (i, 0)),
    out_shape=jax.ShapeDtypeStruct((NP, D), jnp.float32),
)


def _g_body(acc_ref, h2_ref, degp_ref, b_ref, g_ref, gt_ref):
    deg = degp_ref[0, :] + degp_ref[1, :] + 1.0
    dis = lax.rsqrt(deg)
    t = (acc_ref[0] + acc_ref[1] + h2_ref[...]) * dis[:, None] + b_ref[...]
    gb = jnp.maximum(t, 0.0).astype(jnp.bfloat16)
    g_ref[...] = gb
    gt_ref[...] = gb.T


_g_call = pl.pallas_call(
    _g_body,
    grid=(NP // _RB2,),
    in_specs=[
        pl.BlockSpec((NC, _RB2, D), lambda i: (0, i, 0)),
        pl.BlockSpec((_RB2, D), lambda i: (i, 0)),
        pl.BlockSpec((NC, _RB2), lambda i: (0, i)),
        pl.BlockSpec((1, D), lambda i: (0, 0)),
    ],
    out_specs=[
        pl.BlockSpec((_RB2, D), lambda i: (i, 0)),
        pl.BlockSpec((D, _RB2), lambda i: (0, i)),
    ],
    out_shape=[
        jax.ShapeDtypeStruct((NP, D), jnp.bfloat16),
        jax.ShapeDtypeStruct((128, N_NODES), jnp.bfloat16),
    ],
)


_MB = 400


def _mm_body(gr_ref, gt_ref, out_ref):
    out_ref[...] = jnp.dot(gr_ref[...], gt_ref[...], preferred_element_type=jnp.float32)


_mm_call = pl.pallas_call(
    _mm_body,
    grid=(N_NODES // _MB,),
    in_specs=[
        pl.BlockSpec((_MB, D), lambda i: (i, 0)),
        pl.BlockSpec((D, N_NODES), lambda i: (0, 0)),
    ],
    out_specs=pl.BlockSpec((_MB, N_NODES), lambda i: (i, 0)),
    out_shape=jax.ShapeDtypeStruct((N_NODES, N_NODES), jnp.float32),
)


def kernel(z, edge_index, W, b):
    pad = (N_NODES + (jnp.arange(EP - N_EDGES) % (NP - N_NODES))).astype(jnp.int32)
    cat = jnp.concatenate(
        [edge_index, jnp.broadcast_to(pad, (2, EP - N_EDGES))], axis=1)
    eip64 = cat.reshape(2, _BROWS, _BW)
    eip128 = cat.reshape(2, DROWS, 128)
    zp = jnp.zeros((NP, D), jnp.float32).at[:N_NODES].set(z)

    h = _zw_call(zp, W)
    degp = _deg_call(eip128)
    h2 = _scale_call(h, degp)
    accp = _agg_call(eip64, h2)
    g, gt = _g_call(accp, h2, degp, b.reshape(1, D))
    return _mm_call(g, gt)


# speedup vs baseline: 26.2082x; 1.0176x over previous
import functools

import jax
import jax.numpy as jnp
from jax import lax
from jax.experimental import pallas as pl
from jax.experimental.pallas import tpu as pltpu
from jax.experimental.pallas import tpu_sc as plsc

NC, NS, L = 2, 16, 16
N_NODES = 10000
D = 128
N_EDGES = 320000
NP = 10240
EP = 327680
NPT = NP // NS
DROWS = EP // 128
DRPW = DROWS // (NC * NS)
_BW = 64
_BROWS = EP // _BW
_BPW = _BROWS // (NC * NS)
_CH = 32
_NCH = _BPW // _CH
_RING = 4

_sc_mesh = plsc.VectorSubcoreMesh(core_axis_name="c", subcore_axis_name="s")


def _deg_body(ei_hbm, deg_hbm, idx_v, ones_v, buf_v, deg_sh, sem):
    c = lax.axis_index("c")
    s = lax.axis_index("s")
    row0 = (c * NS + s) * _BPW
    for i in range(_BW // 16):
        ones_v[pl.ds(i * 16, 16)] = jnp.ones((16,), jnp.float32)
    for i in range(NPT // 16):
        buf_v[pl.ds(i * 16, 16)] = jnp.zeros((16,), jnp.float32)
    pltpu.sync_copy(buf_v, deg_sh.at[pl.ds(s * NPT, NPT)])
    pltpu.sync_copy(ei_hbm.at[1, pl.ds(row0, _BPW)], idx_v)
    plsc.subcore_barrier()

    def step(j, carry):
        pltpu.sync_copy(ones_v, deg_sh.at[idx_v.at[j]], add=True)
        return carry

    lax.fori_loop(0, _BPW, step, 0)
    plsc.subcore_barrier()
    pltpu.sync_copy(deg_sh.at[pl.ds(s * NPT, NPT)], buf_v)
    pltpu.sync_copy(buf_v, deg_hbm.at[c, pl.ds(s * NPT, NPT)])


_deg_call = pl.kernel(
    _deg_body,
    out_type=jax.ShapeDtypeStruct((NC, NP), jnp.float32),
    mesh=_sc_mesh,
    scratch_types=[
        pltpu.VMEM((_BPW, _BW), jnp.int32),
        pltpu.VMEM((_BW,), jnp.float32),
        pltpu.VMEM((NPT,), jnp.float32),
        pltpu.VMEM_SHARED((NP,), jnp.float32),
        pltpu.SemaphoreType.DMA,
    ],
)


def _agg_body(ei_hbm, h2_hbm, acc_hbm, sidx, didx,
              r0, r1, r2, r3, acc_sh, s0, s1, s2, s3):
    c = lax.axis_index("c")
    s = lax.axis_index("s")
    rows = [r0, r1, r2, r3]
    sems = [s0, s1, s2, s3]
    row0 = (c * NS + s) * _BPW
    pltpu.sync_copy(h2_hbm.at[pl.ds(NP - 64, 64)], r3)
    for t in range(NPT // 64):
        pltpu.sync_copy(r3, acc_sh.at[pl.ds(s * NPT + t * 64, 64)])
    plsc.subcore_barrier()

    for ch in range(_NCH):
        pltpu.sync_copy(ei_hbm.at[0, pl.ds(row0 + ch * _CH, _CH)], sidx)
        pltpu.sync_copy(ei_hbm.at[1, pl.ds(row0 + ch * _CH, _CH)], didx)
        for q in range(_RING - 1):
            pltpu.async_copy(h2_hbm.at[sidx.at[q]], rows[q], sems[q])

        def grp(p, carry):
            for q in range(_RING):
                i = _RING * p + q
                nxt = jnp.minimum(i + _RING - 1, _CH - 1)
                pltpu.async_copy(
                    h2_hbm.at[sidx.at[nxt]], rows[(q + 3) % 4], sems[(q + 3) % 4])
                pltpu.make_async_copy(h2_hbm.at[sidx.at[0]], rows[q], sems[q]).wait()
                pltpu.sync_copy(rows[q], acc_sh.at[didx.at[i]], add=True)
            return carry

        lax.fori_loop(0, _CH // _RING, grp, 0)
        for q in range(_RING - 1):
            pltpu.make_async_copy(h2_hbm.at[sidx.at[0]], rows[q], sems[q]).wait()
    plsc.subcore_barrier()
    for t in range(NPT // 64):
        pltpu.sync_copy(acc_sh.at[pl.ds(s * NPT + t * 64, 64)], r0)
        pltpu.sync_copy(r0, acc_hbm.at[c, pl.ds(s * NPT + t * 64, 64)])


_agg_call = pl.kernel(
    _agg_body,
    out_type=jax.ShapeDtypeStruct((NC, NP, D), jnp.float32),
    mesh=_sc_mesh,
    scratch_types=[
        pltpu.VMEM((_CH, _BW), jnp.int32),
        pltpu.VMEM((_CH, _BW), jnp.int32),
        pltpu.VMEM((_BW, D), jnp.float32),
        pltpu.VMEM((_BW, D), jnp.float32),
        pltpu.VMEM((_BW, D), jnp.float32),
        pltpu.VMEM((_BW, D), jnp.float32),
        pltpu.VMEM_SHARED((NP, D), jnp.float32),
        pltpu.SemaphoreType.DMA,
        pltpu.SemaphoreType.DMA,
        pltpu.SemaphoreType.DMA,
        pltpu.SemaphoreType.DMA,
    ],
)


_RB2 = 2048


def _zw_body(z_ref, w_ref, h_ref):
    h_ref[...] = jnp.dot(z_ref[...], w_ref[...], preferred_element_type=jnp.float32)


_zw_call = pl.pallas_call(
    _zw_body,
    grid=(NP // _RB2,),
    in_specs=[
        pl.BlockSpec((_RB2, D), lambda i: (i, 0)),
        pl.BlockSpec((D, D), lambda i: (0, 0)),
    ],
    out_specs=pl.BlockSpec((_RB2, D), lambda i: (i, 0)),
    out_shape=jax.ShapeDtypeStruct((NP, D), jnp.float32),
)


def _scale_body(h_ref, degp_ref, h2_ref):
    i = pl.program_id(0)
    deg = degp_ref[0, :] + degp_ref[1, :] + 1.0
    dis = lax.rsqrt(deg)
    row = i * _RB2 + lax.broadcasted_iota(jnp.int32, (_RB2, 1), 0)
    val = h_ref[...] * dis[:, None]
    h2_ref[...] = jnp.where(row < N_NODES, val, 0.0)


_scale_call = pl.pallas_call(
    _scale_body,
    grid=(NP // _RB2,),
    in_specs=[
        pl.BlockSpec((_RB2, D), lambda i: (i, 0)),
        pl.BlockSpec((NC, _RB2), lambda i: (0, i)),
    ],
    out_specs=pl.BlockSpec((_RB2, D), lambda i: (i, 0)),
    out_shape=jax.ShapeDtypeStruct((NP, D), jnp.float32),
)


def _g_body(acc_ref, h2_ref, degp_ref, b_ref, g_ref, gt_ref):
    deg = degp_ref[0, :] + degp_ref[1, :] + 1.0
    dis = lax.rsqrt(deg)
    t = (acc_ref[0] + acc_ref[1] + h2_ref[...]) * dis[:, None] + b_ref[...]
    gb = jnp.maximum(t, 0.0).astype(jnp.bfloat16)
    g_ref[...] = gb
    gt_ref[...] = gb.T


_g_call = pl.pallas_call(
    _g_body,
    grid=(NP // _RB2,),
    in_specs=[
        pl.BlockSpec((NC, _RB2, D), lambda i: (0, i, 0)),
        pl.BlockSpec((_RB2, D), lambda i: (i, 0)),
        pl.BlockSpec((NC, _RB2), lambda i: (0, i)),
        pl.BlockSpec((1, D), lambda i: (0, 0)),
    ],
    out_specs=[
        pl.BlockSpec((_RB2, D), lambda i: (i, 0)),
        pl.BlockSpec((D, _RB2), lambda i: (0, i)),
    ],
    out_shape=[
        jax.ShapeDtypeStruct((NP, D), jnp.bfloat16),
        jax.ShapeDtypeStruct((128, N_NODES), jnp.bfloat16),
    ],
)


_MB = 400


def _mm_body(gr_ref, gt_ref, out_ref):
    out_ref[...] = jnp.dot(gr_ref[...], gt_ref[...], preferred_element_type=jnp.float32)


_mm_call = pl.pallas_call(
    _mm_body,
    grid=(N_NODES // _MB,),
    in_specs=[
        pl.BlockSpec((_MB, D), lambda i: (i, 0)),
        pl.BlockSpec((D, N_NODES), lambda i: (0, 0)),
    ],
    out_specs=pl.BlockSpec((_MB, N_NODES), lambda i: (i, 0)),
    out_shape=jax.ShapeDtypeStruct((N_NODES, N_NODES), jnp.float32),
)


def kernel(z, edge_index, W, b):
    pad = (N_NODES + (jnp.arange(EP - N_EDGES) % (NP - N_NODES))).astype(jnp.int32)
    cat = jnp.concatenate(
        [edge_index, jnp.broadcast_to(pad, (2, EP - N_EDGES))], axis=1)
    eip64 = cat.reshape(2, _BROWS, _BW)

    h = _zw_call(z, W)
    degp = _deg_call(eip64)
    h2 = _scale_call(h, degp)
    accp = _agg_call(eip64, h2)
    g, gt = _g_call(accp, h2, degp, b.reshape(1, D))
    return _mm_call(g, gt)


# speedup vs baseline: 27.3061x; 1.0419x over previous
import functools

import jax
import jax.numpy as jnp
from jax import lax
from jax.experimental import pallas as pl
from jax.experimental.pallas import tpu as pltpu
from jax.experimental.pallas import tpu_sc as plsc

NC, NS, L = 2, 16, 16
N_NODES = 10000
D = 128
N_EDGES = 320000
NP = 10240
EP = 327680
NPT = NP // NS
DROWS = EP // 128
DRPW = DROWS // (NC * NS)
_BW = 64
_BROWS = EP // _BW
_BPW = _BROWS // (NC * NS)
_CH = 32
_NCH = _BPW // _CH
_RING = 4

_sc_mesh = plsc.VectorSubcoreMesh(core_axis_name="c", subcore_axis_name="s")


def _deg_body(ei_hbm, deg_hbm, idx_v, ones_v, buf_v, deg_sh, sem, sem2):
    c = lax.axis_index("c")
    s = lax.axis_index("s")
    row0 = (c * NS + s) * _BPW
    for i in range(_BW // 16):
        ones_v[pl.ds(i * 16, 16)] = jnp.ones((16,), jnp.float32)
    for i in range(NPT // 16):
        buf_v[pl.ds(i * 16, 16)] = jnp.zeros((16,), jnp.float32)
    pltpu.sync_copy(buf_v, deg_sh.at[pl.ds(s * NPT, NPT)])
    pltpu.sync_copy(ei_hbm.at[1, pl.ds(row0, _BPW)], idx_v)
    plsc.subcore_barrier()

    pltpu.async_copy(ones_v, deg_sh.at[idx_v.at[0]], sem, add=True)

    def step(p, carry):
        pltpu.async_copy(ones_v, deg_sh.at[idx_v.at[2 * p + 1]], sem2, add=True)
        pltpu.make_async_copy(ones_v, deg_sh.at[idx_v.at[0]], sem).wait()
        pltpu.async_copy(ones_v, deg_sh.at[idx_v.at[2 * p + 2]], sem, add=True)
        pltpu.make_async_copy(ones_v, deg_sh.at[idx_v.at[0]], sem2).wait()
        return carry

    lax.fori_loop(0, _BPW // 2 - 1, step, 0)
    pltpu.async_copy(ones_v, deg_sh.at[idx_v.at[_BPW - 1]], sem2, add=True)
    pltpu.make_async_copy(ones_v, deg_sh.at[idx_v.at[0]], sem).wait()
    pltpu.make_async_copy(ones_v, deg_sh.at[idx_v.at[0]], sem2).wait()
    plsc.subcore_barrier()
    pltpu.sync_copy(deg_sh.at[pl.ds(s * NPT, NPT)], buf_v)
    pltpu.sync_copy(buf_v, deg_hbm.at[c, pl.ds(s * NPT, NPT)])


_deg_call = pl.kernel(
    _deg_body,
    out_type=jax.ShapeDtypeStruct((NC, NP), jnp.float32),
    mesh=_sc_mesh,
    scratch_types=[
        pltpu.VMEM((_BPW, _BW), jnp.int32),
        pltpu.VMEM((_BW,), jnp.float32),
        pltpu.VMEM((NPT,), jnp.float32),
        pltpu.VMEM_SHARED((NP,), jnp.float32),
        pltpu.SemaphoreType.DMA,
        pltpu.SemaphoreType.DMA,
    ],
)


def _agg_body(ei_hbm, h2_hbm, acc_hbm, sidx, didx,
              r0, r1, r2, r3, acc_sh, s0, s1, s2, s3):
    c = lax.axis_index("c")
    s = lax.axis_index("s")
    rows = [r0, r1, r2, r3]
    sems = [s0, s1, s2, s3]
    row0 = (c * NS + s) * _BPW
    pltpu.sync_copy(h2_hbm.at[pl.ds(NP - 64, 64)], r3)
    for t in range(NPT // 64):
        pltpu.sync_copy(r3, acc_sh.at[pl.ds(s * NPT + t * 64, 64)])
    plsc.subcore_barrier()

    for ch in range(_NCH):
        pltpu.sync_copy(ei_hbm.at[0, pl.ds(row0 + ch * _CH, _CH)], sidx)
        pltpu.sync_copy(ei_hbm.at[1, pl.ds(row0 + ch * _CH, _CH)], didx)
        for q in range(_RING - 1):
            pltpu.async_copy(h2_hbm.at[sidx.at[q]], rows[q], sems[q])

        def grp(p, carry):
            for q in range(_RING):
                i = _RING * p + q
                nxt = jnp.minimum(i + _RING - 1, _CH - 1)
                pltpu.async_copy(
                    h2_hbm.at[sidx.at[nxt]], rows[(q + 3) % 4], sems[(q + 3) % 4])
                pltpu.make_async_copy(h2_hbm.at[sidx.at[0]], rows[q], sems[q]).wait()
                pltpu.sync_copy(rows[q], acc_sh.at[didx.at[i]], add=True)
            return carry

        lax.fori_loop(0, _CH // _RING, grp, 0)
        for q in range(_RING - 1):
            pltpu.make_async_copy(h2_hbm.at[sidx.at[0]], rows[q], sems[q]).wait()
    plsc.subcore_barrier()
    for t in range(NPT // 64):
        pltpu.sync_copy(acc_sh.at[pl.ds(s * NPT + t * 64, 64)], r0)
        pltpu.sync_copy(r0, acc_hbm.at[c, pl.ds(s * NPT + t * 64, 64)])


_agg_call = pl.kernel(
    _agg_body,
    out_type=jax.ShapeDtypeStruct((NC, NP, D), jnp.float32),
    mesh=_sc_mesh,
    scratch_types=[
        pltpu.VMEM((_CH, _BW), jnp.int32),
        pltpu.VMEM((_CH, _BW), jnp.int32),
        pltpu.VMEM((_BW, D), jnp.float32),
        pltpu.VMEM((_BW, D), jnp.float32),
        pltpu.VMEM((_BW, D), jnp.float32),
        pltpu.VMEM((_BW, D), jnp.float32),
        pltpu.VMEM_SHARED((NP, D), jnp.float32),
        pltpu.SemaphoreType.DMA,
        pltpu.SemaphoreType.DMA,
        pltpu.SemaphoreType.DMA,
        pltpu.SemaphoreType.DMA,
    ],
)


_RB2 = 2048


def _zw_body(z_ref, w_ref, h_ref):
    h_ref[...] = jnp.dot(z_ref[...], w_ref[...], preferred_element_type=jnp.float32)


_zw_call = pl.pallas_call(
    _zw_body,
    grid=(NP // _RB2,),
    in_specs=[
        pl.BlockSpec((_RB2, D), lambda i: (i, 0)),
        pl.BlockSpec((D, D), lambda i: (0, 0)),
    ],
    out_specs=pl.BlockSpec((_RB2, D), lambda i: (i, 0)),
    out_shape=jax.ShapeDtypeStruct((NP, D), jnp.float32),
)


def _scale_body(h_ref, degp_ref, h2_ref):
    i = pl.program_id(0)
    deg = degp_ref[0, :] + degp_ref[1, :] + 1.0
    dis = lax.rsqrt(deg)
    row = i * _RB2 + lax.broadcasted_iota(jnp.int32, (_RB2, 1), 0)
    val = h_ref[...] * dis[:, None]
    h2_ref[...] = jnp.where(row < N_NODES, val, 0.0)


_scale_call = pl.pallas_call(
    _scale_body,
    grid=(NP // _RB2,),
    in_specs=[
        pl.BlockSpec((_RB2, D), lambda i: (i, 0)),
        pl.BlockSpec((NC, _RB2), lambda i: (0, i)),
    ],
    out_specs=pl.BlockSpec((_RB2, D), lambda i: (i, 0)),
    out_shape=jax.ShapeDtypeStruct((NP, D), jnp.float32),
)


def _g_body(acc_ref, h2_ref, degp_ref, b_ref, g_ref, gt_ref):
    deg = degp_ref[0, :] + degp_ref[1, :] + 1.0
    dis = lax.rsqrt(deg)
    t = (acc_ref[0] + acc_ref[1] + h2_ref[...]) * dis[:, None] + b_ref[...]
    gb = jnp.maximum(t, 0.0).astype(jnp.bfloat16)
    g_ref[...] = gb
    gt_ref[...] = gb.T


_g_call = pl.pallas_call(
    _g_body,
    grid=(NP // _RB2,),
    in_specs=[
        pl.BlockSpec((NC, _RB2, D), lambda i: (0, i, 0)),
        pl.BlockSpec((_RB2, D), lambda i: (i, 0)),
        pl.BlockSpec((NC, _RB2), lambda i: (0, i)),
        pl.BlockSpec((1, D), lambda i: (0, 0)),
    ],
    out_specs=[
        pl.BlockSpec((_RB2, D), lambda i: (i, 0)),
        pl.BlockSpec((D, _RB2), lambda i: (0, i)),
    ],
    out_shape=[
        jax.ShapeDtypeStruct((NP, D), jnp.bfloat16),
        jax.ShapeDtypeStruct((128, N_NODES), jnp.bfloat16),
    ],
)


_MB = 400


def _mm_body(gr_ref, gt_ref, out_ref):
    out_ref[...] = jnp.dot(gr_ref[...], gt_ref[...], preferred_element_type=jnp.float32)


_mm_call = pl.pallas_call(
    _mm_body,
    grid=(N_NODES // _MB,),
    in_specs=[
        pl.BlockSpec((_MB, D), lambda i: (i, 0)),
        pl.BlockSpec((D, N_NODES), lambda i: (0, 0)),
    ],
    out_specs=pl.BlockSpec((_MB, N_NODES), lambda i: (i, 0)),
    out_shape=jax.ShapeDtypeStruct((N_NODES, N_NODES), jnp.float32),
)


def kernel(z, edge_index, W, b):
    pad = (N_NODES + (jnp.arange(EP - N_EDGES) % (NP - N_NODES))).astype(jnp.int32)
    eip64 = jnp.concatenate(
        [edge_index.reshape(2, N_EDGES // _BW, _BW),
         jnp.broadcast_to(pad.reshape(1, -1, _BW), (2, (EP - N_EDGES) // _BW, _BW))],
        axis=1)

    h = _zw_call(z, W)
    degp = _deg_call(eip64)
    h2 = _scale_call(h, degp)
    accp = _agg_call(eip64, h2)
    g, gt = _g_call(accp, h2, degp, b.reshape(1, D))
    return _mm_call(g, gt)
